# baseline (speedup 1.0000x reference)
import functools

import jax
import jax.numpy as jnp
from jax.experimental import pallas as pl
from jax.experimental.pallas import tpu as pltpu

EPS = 1e-5
H = W = 32
HP = H + 2
PREC = jax.lax.Precision.HIGHEST


def _conv1_body(xp_ref, w_ref, b_ref, y_ref, st_ref):
    cin, cout = 384, 256
    acc = jnp.zeros((H * W, cout), jnp.float32)
    for t in range(9):
        dy, dx = divmod(t, 3)
        xs = xp_ref[0, dy:dy + H, dx:dx + W, :].reshape(H * W, cin)
        acc = acc + jnp.dot(xs, w_ref[t], preferred_element_type=jnp.float32,
                            precision=PREC)
    y = acc + b_ref[0]
    y_ref[0] = y.reshape(H, W, cout)
    st_ref[0, 0] = jnp.sum(y, axis=0)
    st_ref[0, 1] = jnp.sum(y * y, axis=0)


def _bn_affine(st_ref, g_ref, be_ref, n):
    mean = jnp.sum(st_ref[:, 0, :], axis=0) / n
    sq = jnp.sum(st_ref[:, 1, :], axis=0) / n
    var = sq - mean * mean
    scale = g_ref[0] / jnp.sqrt(var + EPS)
    shift = be_ref[0] - mean * scale
    return scale, shift


def _conv2_body(y1_ref, st_ref, g_ref, be_ref, w_ref, b_ref,
                y2_ref, st2_ref, zp_ref):
    c = 256
    scale, shift = _bn_affine(st_ref, g_ref, be_ref, 16.0 * H * W)
    z = jnp.maximum(y1_ref[0] * scale + shift, 0.0)
    zp_ref[...] = jnp.zeros((HP, HP, c), jnp.float32)
    zp_ref[1:1 + H, 1:1 + W, :] = z
    acc = jnp.zeros((H * W, c), jnp.float32)
    for t in range(9):
        dy, dx = divmod(t, 3)
        zs = zp_ref[dy:dy + H, dx:dx + W, :].reshape(H * W, c)
        acc = acc + jnp.dot(zs, w_ref[t], preferred_element_type=jnp.float32,
                            precision=PREC)
    y = acc + b_ref[0]
    y2_ref[0] = y.reshape(H, W, c)
    st2_ref[0, 0] = jnp.sum(y, axis=0)
    st2_ref[0, 1] = jnp.sum(y * y, axis=0)


def _conv3_body(y2_ref, st_ref, g_ref, be_ref, w_ref, b_ref, out_ref, zp_ref):
    c = 256
    scale, shift = _bn_affine(st_ref, g_ref, be_ref, 16.0 * H * W)
    z = jnp.maximum(y2_ref[0] * scale + shift, 0.0)
    zp_ref[...] = jnp.zeros((HP, HP, c), jnp.float32)
    zp_ref[1:1 + H, 1:1 + W, :] = z
    acc = jnp.zeros((H * W, 128), jnp.float32)
    for t in range(9):
        dy, dx = divmod(t, 3)
        zs = zp_ref[dy:dy + H, dx:dx + W, :].reshape(H * W, c)
        acc = acc + jnp.dot(zs, w_ref[t], preferred_element_type=jnp.float32,
                            precision=PREC)
    y = acc[:, 0:1] + b_ref[0]
    out_ref[0] = jax.nn.sigmoid(y)


@jax.jit
def kernel(dino_features, W1, b1, g1, be1, W2, b2, g2, be2, W3, b3):
    B = dino_features.shape[0]
    f32 = jnp.float32

    xp = jnp.pad(dino_features, ((0, 0), (1, 1), (1, 1), (0, 0)))
    w1r = jnp.transpose(W1, (2, 3, 1, 0)).reshape(9, 384, 256)
    w2r = jnp.transpose(W2, (2, 3, 1, 0)).reshape(9, 256, 256)
    w3r = jnp.transpose(W3, (2, 3, 1, 0)).reshape(9, 256, 1)
    w3r = jnp.pad(w3r, ((0, 0), (0, 0), (0, 127)))

    full = lambda shape: pl.BlockSpec(shape, lambda b: (0,) * len(shape))
    img = lambda shape: pl.BlockSpec(shape, lambda b: (b,) + (0,) * (len(shape) - 1))

    y1, st1 = pl.pallas_call(
        _conv1_body,
        grid=(B,),
        in_specs=[img((1, HP, HP, 384)), full((9, 384, 256)), full((1, 256))],
        out_specs=[img((1, H, W, 256)), img((1, 2, 256))],
        out_shape=[jax.ShapeDtypeStruct((B, H, W, 256), f32),
                   jax.ShapeDtypeStruct((B, 2, 256), f32)],
        compiler_params=pltpu.CompilerParams(
            dimension_semantics=("arbitrary",)),
    )(xp, w1r, b1.reshape(1, 256))

    y2, st2 = pl.pallas_call(
        _conv2_body,
        grid=(B,),
        in_specs=[img((1, H, W, 256)), full((B, 2, 256)), full((1, 256)),
                  full((1, 256)), full((9, 256, 256)), full((1, 256))],
        out_specs=[img((1, H, W, 256)), img((1, 2, 256))],
        out_shape=[jax.ShapeDtypeStruct((B, H, W, 256), f32),
                   jax.ShapeDtypeStruct((B, 2, 256), f32)],
        scratch_shapes=[pltpu.VMEM((HP, HP, 256), f32)],
        compiler_params=pltpu.CompilerParams(
            dimension_semantics=("arbitrary",)),
    )(y1, st1, g1.reshape(1, 256), be1.reshape(1, 256), w2r,
      b2.reshape(1, 256))

    out = pl.pallas_call(
        _conv3_body,
        grid=(B,),
        in_specs=[img((1, H, W, 256)), full((B, 2, 256)), full((1, 256)),
                  full((1, 256)), full((9, 256, 128)), full((1, 1))],
        out_specs=img((1, H * W, 1)),
        out_shape=jax.ShapeDtypeStruct((B, H * W, 1), f32),
        scratch_shapes=[pltpu.VMEM((HP, HP, 256), f32)],
        compiler_params=pltpu.CompilerParams(
            dimension_semantics=("arbitrary",)),
    )(y2, st2, g2.reshape(1, 256), be2.reshape(1, 256), w3r,
      b3.reshape(1, 1))

    return out.reshape(B, H, W, 1)


# speedup vs baseline: 3.7598x; 3.7598x over previous
import functools

import jax
import jax.numpy as jnp
from jax.experimental import pallas as pl
from jax.experimental.pallas import tpu as pltpu

EPS = 1e-5
H = W = 32
HP = H + 2
PREC = jax.lax.Precision.DEFAULT


def _conv1_body(xp_ref, w_ref, b_ref, y_ref, st_ref):
    cin, cout = 384, 256
    acc = jnp.zeros((H * W, cout), jnp.float32)
    for t in range(9):
        dy, dx = divmod(t, 3)
        xs = xp_ref[0, dy:dy + H, dx:dx + W, :].reshape(H * W, cin)
        acc = acc + jnp.dot(xs, w_ref[t], preferred_element_type=jnp.float32,
                            precision=PREC)
    y = acc + b_ref[0]
    y_ref[0] = y.reshape(H, W, cout)
    st_ref[0, 0] = jnp.sum(y, axis=0)
    st_ref[0, 1] = jnp.sum(y * y, axis=0)


def _bn_affine(st_ref, g_ref, be_ref, n):
    mean = jnp.sum(st_ref[:, 0, :], axis=0) / n
    sq = jnp.sum(st_ref[:, 1, :], axis=0) / n
    var = sq - mean * mean
    scale = g_ref[0] / jnp.sqrt(var + EPS)
    shift = be_ref[0] - mean * scale
    return scale, shift


def _conv2_body(y1_ref, st_ref, g_ref, be_ref, w_ref, b_ref,
                y2_ref, st2_ref, zp_ref):
    c = 256
    scale, shift = _bn_affine(st_ref, g_ref, be_ref, 16.0 * H * W)
    z = jnp.maximum(y1_ref[0] * scale + shift, 0.0)
    zp_ref[...] = jnp.zeros((HP, HP, c), jnp.float32)
    zp_ref[1:1 + H, 1:1 + W, :] = z
    acc = jnp.zeros((H * W, c), jnp.float32)
    for t in range(9):
        dy, dx = divmod(t, 3)
        zs = zp_ref[dy:dy + H, dx:dx + W, :].reshape(H * W, c)
        acc = acc + jnp.dot(zs, w_ref[t], preferred_element_type=jnp.float32,
                            precision=PREC)
    y = acc + b_ref[0]
    y2_ref[0] = y.reshape(H, W, c)
    st2_ref[0, 0] = jnp.sum(y, axis=0)
    st2_ref[0, 1] = jnp.sum(y * y, axis=0)


def _conv3_body(y2_ref, st_ref, g_ref, be_ref, w_ref, b_ref, out_ref, zp_ref):
    c = 256
    scale, shift = _bn_affine(st_ref, g_ref, be_ref, 16.0 * H * W)
    z = jnp.maximum(y2_ref[0] * scale + shift, 0.0)
    zp_ref[...] = jnp.zeros((HP, HP, c), jnp.float32)
    zp_ref[1:1 + H, 1:1 + W, :] = z
    acc = jnp.zeros((H * W, 128), jnp.float32)
    for t in range(9):
        dy, dx = divmod(t, 3)
        zs = zp_ref[dy:dy + H, dx:dx + W, :].reshape(H * W, c)
        acc = acc + jnp.dot(zs, w_ref[t], preferred_element_type=jnp.float32,
                            precision=PREC)
    y = acc[:, 0:1] + b_ref[0]
    out_ref[0] = jax.nn.sigmoid(y)


@jax.jit
def kernel(dino_features, W1, b1, g1, be1, W2, b2, g2, be2, W3, b3):
    B = dino_features.shape[0]
    f32 = jnp.float32

    xp = jnp.pad(dino_features, ((0, 0), (1, 1), (1, 1), (0, 0)))
    w1r = jnp.transpose(W1, (2, 3, 1, 0)).reshape(9, 384, 256)
    w2r = jnp.transpose(W2, (2, 3, 1, 0)).reshape(9, 256, 256)
    w3r = jnp.transpose(W3, (2, 3, 1, 0)).reshape(9, 256, 1)
    w3r = jnp.pad(w3r, ((0, 0), (0, 0), (0, 127)))

    full = lambda shape: pl.BlockSpec(shape, lambda b: (0,) * len(shape))
    img = lambda shape: pl.BlockSpec(shape, lambda b: (b,) + (0,) * (len(shape) - 1))

    y1, st1 = pl.pallas_call(
        _conv1_body,
        grid=(B,),
        in_specs=[img((1, HP, HP, 384)), full((9, 384, 256)), full((1, 256))],
        out_specs=[img((1, H, W, 256)), img((1, 2, 256))],
        out_shape=[jax.ShapeDtypeStruct((B, H, W, 256), f32),
                   jax.ShapeDtypeStruct((B, 2, 256), f32)],
        compiler_params=pltpu.CompilerParams(
            dimension_semantics=("arbitrary",)),
    )(xp, w1r, b1.reshape(1, 256))

    y2, st2 = pl.pallas_call(
        _conv2_body,
        grid=(B,),
        in_specs=[img((1, H, W, 256)), full((B, 2, 256)), full((1, 256)),
                  full((1, 256)), full((9, 256, 256)), full((1, 256))],
        out_specs=[img((1, H, W, 256)), img((1, 2, 256))],
        out_shape=[jax.ShapeDtypeStruct((B, H, W, 256), f32),
                   jax.ShapeDtypeStruct((B, 2, 256), f32)],
        scratch_shapes=[pltpu.VMEM((HP, HP, 256), f32)],
        compiler_params=pltpu.CompilerParams(
            dimension_semantics=("arbitrary",)),
    )(y1, st1, g1.reshape(1, 256), be1.reshape(1, 256), w2r,
      b2.reshape(1, 256))

    out = pl.pallas_call(
        _conv3_body,
        grid=(B,),
        in_specs=[img((1, H, W, 256)), full((B, 2, 256)), full((1, 256)),
                  full((1, 256)), full((9, 256, 128)), full((1, 1))],
        out_specs=img((1, H * W, 1)),
        out_shape=jax.ShapeDtypeStruct((B, H * W, 1), f32),
        scratch_shapes=[pltpu.VMEM((HP, HP, 256), f32)],
        compiler_params=pltpu.CompilerParams(
            dimension_semantics=("arbitrary",)),
    )(y2, st2, g2.reshape(1, 256), be2.reshape(1, 256), w3r,
      b3.reshape(1, 1))

    return out.reshape(B, H, W, 1)
